# Initial kernel scaffold; baseline (speedup 1.0000x reference)
import jax
import jax.numpy as jnp
from jax.experimental import pallas as pl


def kernel(tokens, embeddings):
    raise NotImplementedError("write your pallas kernel here")


# speedup vs baseline: 2.9694x; 2.9694x over previous
import functools

import jax
import jax.numpy as jnp
from jax import lax
from jax.experimental import pallas as pl
from jax.experimental.pallas import tpu as pltpu
from jax.experimental.pallas import tpu_sc as plsc

_INFO = plsc.get_sparse_core_info()
_NC = _INFO.num_cores
_NS = _INFO.num_subcores
_NW = _NC * _NS
_L = 128


@functools.partial(jax.jit, static_argnums=(2, 3))
def _gather_rows(table, idx, per_w, d):
    n = idx.shape[0]
    n_dma = per_w // _L
    mesh = plsc.VectorSubcoreMesh(core_axis_name="c", subcore_axis_name="s")

    @functools.partial(
        pl.kernel,
        out_type=jax.ShapeDtypeStruct((n, d), jnp.float32),
        mesh=mesh,
        scratch_types=[
            pltpu.VMEM((per_w,), jnp.int32),
            pltpu.VMEM((_L, d), jnp.float32),
            pltpu.SemaphoreType.DMA,
        ],
    )
    def k(table_hbm, idx_hbm, out_hbm, idx_v, rows_v, sem):
        wid = lax.axis_index("s") * _NC + lax.axis_index("c")
        base = wid * per_w
        pltpu.sync_copy(idx_hbm.at[pl.ds(base, per_w)], idx_v)

        @pl.loop(0, n_dma)
        def _(j):
            pltpu.async_copy(
                table_hbm.at[idx_v.at[pl.ds(j * _L, _L)]], rows_v, sem
            ).wait()
            pltpu.sync_copy(rows_v, out_hbm.at[pl.ds(base + j * _L, _L)])

    return k(table, idx)


def kernel(tokens, embeddings):
    b0, b1 = tokens.shape
    d = embeddings.shape[1]
    idx = tokens.reshape(-1).astype(jnp.int32)
    per_w = idx.shape[0] // _NW
    out = _gather_rows(embeddings, idx, per_w, d)
    return out.reshape(b0, b1, d)


# speedup vs baseline: 3.3186x; 1.1176x over previous
import functools

import jax
import jax.numpy as jnp
from jax import lax
from jax.experimental import pallas as pl
from jax.experimental.pallas import tpu as pltpu
from jax.experimental.pallas import tpu_sc as plsc

_INFO = plsc.get_sparse_core_info()
_NC = _INFO.num_cores
_NS = _INFO.num_subcores
_NW = _NC * _NS
_L = 128


@functools.partial(jax.jit, static_argnums=(2, 3))
def _gather_rows(table, idx, per_w, d):
    n = idx.shape[0]
    n_dma = per_w // _L
    mesh = plsc.VectorSubcoreMesh(core_axis_name="c", subcore_axis_name="s")

    nbuf = 5
    lag = 2

    @functools.partial(
        pl.kernel,
        out_type=jax.ShapeDtypeStruct((n, d), jnp.float32),
        mesh=mesh,
        scratch_types=[
            pltpu.VMEM((per_w,), jnp.int32),
            pltpu.VMEM((nbuf, _L, d), jnp.float32),
            pltpu.SemaphoreType.DMA((nbuf,)),
            pltpu.SemaphoreType.DMA((nbuf,)),
        ],
    )
    def k(table_hbm, idx_hbm, out_hbm, idx_v, rows_v, gsem, ssem):
        wid = lax.axis_index("s") * _NC + lax.axis_index("c")
        base = wid * per_w
        pltpu.sync_copy(idx_hbm.at[pl.ds(base, per_w)], idx_v)

        g_desc, s_desc = {}, {}
        for j in range(n_dma + lag):
            if j < n_dma:
                b = j % nbuf
                if j >= nbuf:
                    s_desc[j - nbuf].wait()
                g_desc[j] = pltpu.async_copy(
                    table_hbm.at[idx_v.at[pl.ds(j * _L, _L)]],
                    rows_v.at[b], gsem.at[b])
            i = j - lag
            if i >= 0:
                bi = i % nbuf
                g_desc[i].wait()
                s_desc[i] = pltpu.async_copy(
                    rows_v.at[bi], out_hbm.at[pl.ds(base + i * _L, _L)],
                    ssem.at[bi])
        for i in range(n_dma - nbuf, n_dma):
            s_desc[i].wait()

    return k(table, idx)


def kernel(tokens, embeddings):
    b0, b1 = tokens.shape
    d = embeddings.shape[1]
    idx = tokens.reshape(-1).astype(jnp.int32)
    per_w = idx.shape[0] // _NW
    out = _gather_rows(embeddings, idx, per_w, d)
    return out.reshape(b0, b1, d)


# speedup vs baseline: 5.7865x; 1.7437x over previous
import functools

import jax
import jax.numpy as jnp
from jax import lax
from jax.experimental import pallas as pl
from jax.experimental.pallas import tpu as pltpu
from jax.experimental.pallas import tpu_sc as plsc

_INFO = plsc.get_sparse_core_info()
_NC = _INFO.num_cores
_NS = _INFO.num_subcores
_NW = _NC * _NS


@jax.jit
def _gather3d(table, tokens):
    b, s = tokens.shape
    d = table.shape[1]
    rows_per_w = b // _NW
    mesh = plsc.VectorSubcoreMesh(core_axis_name="c", subcore_axis_name="s")
    nbuf = 5
    lag = 2

    @functools.partial(
        pl.kernel,
        out_type=jax.ShapeDtypeStruct((b, s, d), jnp.float32),
        mesh=mesh,
        scratch_types=[
            pltpu.VMEM((rows_per_w, s), jnp.int32),
            pltpu.VMEM((nbuf, s, d), jnp.float32),
            pltpu.SemaphoreType.DMA((nbuf,)),
            pltpu.SemaphoreType.DMA((nbuf,)),
        ],
    )
    def k(table_hbm, tok_hbm, out_hbm, idx_v, rows_v, gsem, ssem):
        wid = lax.axis_index("s") * _NC + lax.axis_index("c")
        base = wid * rows_per_w
        pltpu.sync_copy(tok_hbm.at[pl.ds(base, rows_per_w)], idx_v)

        g_desc, s_desc = {}, {}
        for j in range(rows_per_w + lag):
            if j < rows_per_w:
                bf = j % nbuf
                if j >= nbuf:
                    s_desc[j - nbuf].wait()
                g_desc[j] = pltpu.async_copy(
                    table_hbm.at[idx_v.at[j]], rows_v.at[bf], gsem.at[bf])
            i = j - lag
            if i >= 0:
                bi = i % nbuf
                g_desc[i].wait()
                s_desc[i] = pltpu.async_copy(
                    rows_v.at[bi], out_hbm.at[base + i], ssem.at[bi])
        for i in range(rows_per_w - nbuf, rows_per_w):
            s_desc[i].wait()

    return k(table, tokens)


def kernel(tokens, embeddings):
    return _gather3d(embeddings, tokens.astype(jnp.int32))


# speedup vs baseline: 10.3338x; 1.7858x over previous
import functools

import jax
import jax.numpy as jnp
from jax import lax
from jax.experimental import pallas as pl
from jax.experimental.pallas import tpu as pltpu
from jax.experimental.pallas import tpu_sc as plsc

_INFO = plsc.get_sparse_core_info()
_NC = _INFO.num_cores
_NS = _INFO.num_subcores
_NW = _NC * _NS


@jax.jit
def _gather_t(table, tok_t):
    s, b = tok_t.shape
    d = table.shape[1]
    rows_per_w = b // _NW
    mesh = plsc.VectorSubcoreMesh(core_axis_name="c", subcore_axis_name="s")
    nbuf = 5
    lag = 2

    @functools.partial(
        pl.kernel,
        out_type=jax.ShapeDtypeStruct((s, b, d), jnp.float32),
        mesh=mesh,
        scratch_types=[
            pltpu.VMEM((s, rows_per_w), jnp.int32),
            pltpu.VMEM((nbuf, rows_per_w, d), jnp.float32),
            pltpu.SemaphoreType.DMA((nbuf,)),
            pltpu.SemaphoreType.DMA((nbuf,)),
        ],
    )
    def k(table_hbm, tok_hbm, out_hbm, idx_v, rows_v, gsem, ssem):
        wid = lax.axis_index("s") * _NC + lax.axis_index("c")
        base = wid * rows_per_w
        pltpu.sync_copy(tok_hbm.at[:, pl.ds(base, rows_per_w)], idx_v)

        g_desc, s_desc = {}, {}
        for j in range(s + lag):
            if j < s:
                bf = j % nbuf
                if j >= nbuf:
                    s_desc[j - nbuf].wait()
                g_desc[j] = pltpu.async_copy(
                    table_hbm.at[idx_v.at[j]], rows_v.at[bf], gsem.at[bf])
            i = j - lag
            if i >= 0:
                bi = i % nbuf
                g_desc[i].wait()
                s_desc[i] = pltpu.async_copy(
                    rows_v.at[bi], out_hbm.at[i, pl.ds(base, rows_per_w)],
                    ssem.at[bi])
        for i in range(s - nbuf, s):
            s_desc[i].wait()

    return k(table, tok_t)


def kernel(tokens, embeddings):
    out_t = _gather_t(embeddings, tokens.T.astype(jnp.int32))
    return out_t.transpose(1, 0, 2)


# speedup vs baseline: 10.4104x; 1.0074x over previous
import functools

import jax
import jax.numpy as jnp
from jax import lax
from jax.experimental import pallas as pl
from jax.experimental.pallas import tpu as pltpu
from jax.experimental.pallas import tpu_sc as plsc

_INFO = plsc.get_sparse_core_info()
_NC = _INFO.num_cores
_NS = _INFO.num_subcores
_NW = _NC * _NS


@jax.jit
def _gather_t(table, tok_t):
    s, b = tok_t.shape
    d = table.shape[1]
    rows_per_w = b // _NW
    mesh = plsc.VectorSubcoreMesh(core_axis_name="c", subcore_axis_name="s")
    nbuf = 7
    lag = 3

    @functools.partial(
        pl.kernel,
        out_type=jax.ShapeDtypeStruct((s, b, d), jnp.float32),
        mesh=mesh,
        scratch_types=[
            pltpu.VMEM((s, rows_per_w), jnp.int32),
            pltpu.VMEM((nbuf, rows_per_w, d), jnp.float32),
            pltpu.SemaphoreType.DMA((nbuf,)),
            pltpu.SemaphoreType.DMA((nbuf,)),
        ],
    )
    def k(table_hbm, tok_hbm, out_hbm, idx_v, rows_v, gsem, ssem):
        wid = lax.axis_index("s") * _NC + lax.axis_index("c")
        base = wid * rows_per_w
        pltpu.sync_copy(tok_hbm.at[:, pl.ds(base, rows_per_w)], idx_v)

        g_desc, s_desc = {}, {}
        for j in range(s + lag):
            if j < s:
                bf = j % nbuf
                if j >= nbuf:
                    s_desc[j - nbuf].wait()
                g_desc[j] = pltpu.async_copy(
                    table_hbm.at[idx_v.at[j]], rows_v.at[bf], gsem.at[bf])
            i = j - lag
            if i >= 0:
                bi = i % nbuf
                g_desc[i].wait()
                s_desc[i] = pltpu.async_copy(
                    rows_v.at[bi], out_hbm.at[i, pl.ds(base, rows_per_w)],
                    ssem.at[bi])
        for i in range(s - nbuf, s):
            s_desc[i].wait()

    return k(table, tok_t)


def kernel(tokens, embeddings):
    out_t = _gather_t(embeddings, tokens.T.astype(jnp.int32))
    return out_t.transpose(1, 0, 2)
